# baseline (speedup 1.0000x reference)
import functools

import numpy as np
import jax
import jax.numpy as jnp
from jax import lax
from jax.experimental import pallas as pl
from jax.experimental.pallas import tpu as pltpu
from jax.experimental.pallas import tpu_sc as plsc

_N = 512
_K = 128
_B = 8
_GROUPS = 2
_ROWS = _B * _GROUPS
_L = 16
_NC = 2
_SLICES = _N // _L

_jk = (np.arange(_N, dtype=np.int64)[:, None] * np.arange(_N, dtype=np.int64)[None, :]) % _N
_theta = 2.0 * np.pi * _jk.astype(np.float64) / _N
_COS = np.cos(_theta).astype(np.float32)
_SIN = np.sin(_theta).astype(np.float32)
_CPS = (_COS + _SIN).astype(np.float32)
_CMS = (_COS - _SIN).astype(np.float32)


def _dot(a, b):
    return jax.lax.dot_general(
        a, b, (((1,), (0,)), ((), ())),
        precision=jax.lax.Precision.DEFAULT,
        preferred_element_type=jnp.float32)


def _energy_kernel(xr_ref, xi_ref, e_ref):
    zr0 = xr_ref[0, 0]
    zi0 = xi_ref[0, 0]
    zr1 = xr_ref[0, 1]
    zi1 = xi_ref[0, 1]
    e_ref[0, 0] = jnp.sum(jnp.abs(jnp.abs(zr0) - jnp.abs(zi1))
                          + jnp.abs(jnp.abs(zr1) - jnp.abs(zi0)),
                          axis=0)


def _sc_mask_body(energy_hbm, mask_hbm, e_v, m_v, sem):
    wid = lax.axis_index("s") * _NC + lax.axis_index("c")

    @pl.when(wid < _ROWS)
    def _():
        pltpu.sync_copy(energy_hbm.at[wid], e_v)
        one_v = jnp.full((_L,), 1, jnp.int32)
        zero_v = jnp.zeros((_L,), jnp.int32)
        lane = lax.iota(jnp.int32, _L)

        def allsum(x):
            for sh in (1, 2, 4, 8):
                x = x + jnp.take(x, lane ^ sh)
            return x

        def prefix_incl(x):
            for sh in (1, 2, 4, 8):
                shifted = jnp.take(x, jnp.maximum(lane - sh, 0))
                x = x + jnp.where(lane >= sh, shifted, zero_v)
            return x

        def count_less(cand):
            def cbody(j, acc):
                v = lax.bitcast_convert_type(e_v[pl.ds(j * _L, _L)],
                                             jnp.int32)
                return acc + jnp.where(v < cand, one_v, zero_v)
            acc = lax.fori_loop(0, _SLICES, cbody, zero_v)
            return allsum(acc)

        def bit_body(k, prefix):
            cand = prefix | lax.shift_left(one_v, jnp.int32(30) - k)
            return jnp.where(count_less(cand) < _K, cand, prefix)
        t = lax.fori_loop(0, 31, bit_body, zero_v)

        remaining = jnp.int32(_K) - count_less(t)

        def mask_body(j, cum):
            sl = pl.ds(j * _L, _L)
            v = lax.bitcast_convert_type(e_v[sl], jnp.int32)
            lt = v < t
            eq = v == t
            eq_i = jnp.where(eq, one_v, zero_v)
            pos = cum + prefix_incl(eq_i)
            take = lt | (eq & (pos <= remaining))
            m_v[sl] = jnp.where(take, jnp.full((_L,), 1.0, jnp.float32),
                                jnp.zeros((_L,), jnp.float32))
            return cum + allsum(eq_i)
        lax.fori_loop(0, _SLICES, mask_body, zero_v)

        pltpu.sync_copy(m_v, mask_hbm.at[wid])


_sc_mask = functools.partial(
    pl.kernel,
    mesh=plsc.VectorSubcoreMesh(core_axis_name="c", subcore_axis_name="s"),
    out_type=jax.ShapeDtypeStruct((_ROWS, _N), jnp.float32),
    scratch_types=[
        pltpu.VMEM((_N,), jnp.float32),
        pltpu.VMEM((_N,), jnp.float32),
        pltpu.SemaphoreType.DMA,
    ],
)(_sc_mask_body)


def _transform_kernel(xr_ref, xi_ref, mask_ref, c_ref, s_ref, cps_ref,
                      cms_ref, or_ref, oi_ref):
    mask = mask_ref[0]
    c = c_ref[...]
    s = s_ref[...]
    cps = cps_ref[...]
    cms = cms_ref[...]
    inv_n = jnp.float32(1.0 / _N)

    for ch in range(2):
        zr = xr_ref[0, ch] * inv_n
        zi = xi_ref[0, ch] * inv_n
        t1 = _dot(c, zr)
        t2 = _dot(s, zi)
        t3 = _dot(cps, zr + zi)
        pr = (t1 - t2) * mask
        pi = (t3 - t1 - t2) * mask
        u1 = _dot(pr, c)
        u2 = _dot(pi, s)
        u3 = _dot(pr + pi, cms)
        or_ref[0, ch] = u1 + u2 + 0.5
        oi_ref[0, ch] = u3 - u1 + u2


@jax.jit
def kernel(Inp_AD_C_real, Inp_AD_C_imag):
    B, C, H, W = Inp_AD_C_real.shape
    G = _GROUPS
    chans = C // G

    img_spec = pl.BlockSpec((1, chans, H, W), lambda b, g: (b, g, 0, 0))
    row_spec = pl.BlockSpec((1, 1, W), lambda b, g: (2 * b + g, 0, 0))
    mat_spec = pl.BlockSpec((_N, _N), lambda b, g: (0, 0))

    energy = pl.pallas_call(
        _energy_kernel,
        grid=(B, G),
        in_specs=[img_spec, img_spec],
        out_specs=row_spec,
        out_shape=jax.ShapeDtypeStruct((_ROWS, 1, W), jnp.float32),
    )(Inp_AD_C_real, Inp_AD_C_imag)

    mask = _sc_mask(energy.reshape(_ROWS, W))

    out_r, out_i = pl.pallas_call(
        _transform_kernel,
        grid=(B, G),
        in_specs=[img_spec, img_spec, row_spec, mat_spec, mat_spec,
                  mat_spec, mat_spec],
        out_specs=[img_spec, img_spec],
        out_shape=[
            jax.ShapeDtypeStruct((B, C, H, W), jnp.float32),
            jax.ShapeDtypeStruct((B, C, H, W), jnp.float32),
        ],
    )(Inp_AD_C_real, Inp_AD_C_imag, mask.reshape(_ROWS, 1, W),
      jnp.asarray(_COS), jnp.asarray(_SIN),
      jnp.asarray(_CPS), jnp.asarray(_CMS))

    return jax.lax.complex(out_r, out_i).astype(jnp.complex64)


# speedup vs baseline: 1.0005x; 1.0005x over previous
import functools

import numpy as np
import jax
import jax.numpy as jnp
from jax import lax
from jax.experimental import pallas as pl
from jax.experimental.pallas import tpu as pltpu
from jax.experimental.pallas import tpu_sc as plsc

_N = 512
_K = 128
_B = 8
_GROUPS = 2
_ROWS = _B * _GROUPS
_L = 16
_NC = 2
_SLICES = _N // _L

_jk = (np.arange(_N, dtype=np.int64)[:, None] * np.arange(_N, dtype=np.int64)[None, :]) % _N
_theta = 2.0 * np.pi * _jk.astype(np.float64) / _N
_COS = np.cos(_theta).astype(np.float32)
_SIN = np.sin(_theta).astype(np.float32)
_CPS = (_COS + _SIN).astype(np.float32)
_CMS = (_COS - _SIN).astype(np.float32)


def _dot(a, b):
    return jax.lax.dot_general(
        a, b, (((1,), (0,)), ((), ())),
        precision=jax.lax.Precision.DEFAULT,
        preferred_element_type=jnp.float32)


def _energy_kernel(xr_ref, xi_ref, e_ref):
    zr0 = xr_ref[0, 0]
    zi0 = xi_ref[0, 0]
    zr1 = xr_ref[0, 1]
    zi1 = xi_ref[0, 1]
    e_ref[0, 0] = jnp.sum(jnp.abs(jnp.abs(zr0) - jnp.abs(zi1))
                          + jnp.abs(jnp.abs(zr1) - jnp.abs(zi0)),
                          axis=0)


def _sc_mask_body(energy_hbm, mask_hbm, e_v, m_v, sem):
    wid = lax.axis_index("s") * _NC + lax.axis_index("c")

    @pl.when(wid < _ROWS)
    def _():
        pltpu.sync_copy(energy_hbm.at[wid, 0], e_v)
        one_v = jnp.full((_L,), 1, jnp.int32)
        zero_v = jnp.zeros((_L,), jnp.int32)
        lane = lax.iota(jnp.int32, _L)

        def allsum(x):
            for sh in (1, 2, 4, 8):
                x = x + jnp.take(x, lane ^ sh)
            return x

        def prefix_incl(x):
            for sh in (1, 2, 4, 8):
                shifted = jnp.take(x, jnp.maximum(lane - sh, 0))
                x = x + jnp.where(lane >= sh, shifted, zero_v)
            return x

        def count_less(cand):
            def cbody(j, acc):
                v = lax.bitcast_convert_type(e_v[pl.ds(j * _L, _L)],
                                             jnp.int32)
                return acc + jnp.where(v < cand, one_v, zero_v)
            acc = lax.fori_loop(0, _SLICES, cbody, zero_v)
            return allsum(acc)

        def bit_body(k, prefix):
            cand = prefix | lax.shift_left(one_v, jnp.int32(30) - k)
            return jnp.where(count_less(cand) < _K, cand, prefix)
        t = lax.fori_loop(0, 31, bit_body, zero_v)

        remaining = jnp.int32(_K) - count_less(t)

        def mask_body(j, cum):
            sl = pl.ds(j * _L, _L)
            v = lax.bitcast_convert_type(e_v[sl], jnp.int32)
            lt = v < t
            eq = v == t
            eq_i = jnp.where(eq, one_v, zero_v)
            pos = cum + prefix_incl(eq_i)
            take = lt | (eq & (pos <= remaining))
            m_v[sl] = jnp.where(take, jnp.full((_L,), 1.0, jnp.float32),
                                jnp.zeros((_L,), jnp.float32))
            return cum + allsum(eq_i)
        lax.fori_loop(0, _SLICES, mask_body, zero_v)

        pltpu.sync_copy(m_v, mask_hbm.at[wid, 0])


_sc_mask = functools.partial(
    pl.kernel,
    mesh=plsc.VectorSubcoreMesh(core_axis_name="c", subcore_axis_name="s"),
    out_type=jax.ShapeDtypeStruct((_ROWS, 1, _N), jnp.float32),
    scratch_types=[
        pltpu.VMEM((_N,), jnp.float32),
        pltpu.VMEM((_N,), jnp.float32),
        pltpu.SemaphoreType.DMA,
    ],
)(_sc_mask_body)


def _transform_kernel(xr_ref, xi_ref, mask_ref, c_ref, s_ref, cps_ref,
                      cms_ref, or_ref, oi_ref):
    mask = mask_ref[0]
    c = c_ref[...]
    s = s_ref[...]
    cps = cps_ref[...]
    cms = cms_ref[...]
    inv_n = jnp.float32(1.0 / _N)

    bf = jnp.bfloat16
    for ch in range(2):
        zr = xr_ref[0, ch] * inv_n
        zi = xi_ref[0, ch] * inv_n
        zr_b = zr.astype(bf)
        zi_b = zi.astype(bf)
        zs_b = (zr + zi).astype(bf)
        t1 = _dot(c, zr_b)
        t2 = _dot(s, zi_b)
        t3 = _dot(cps, zs_b)
        pr = (t1 - t2) * mask
        pi = (t3 - t1 - t2) * mask
        u1 = _dot(pr.astype(bf), c)
        u2 = _dot(pi.astype(bf), s)
        u3 = _dot((pr + pi).astype(bf), cms)
        or_ref[0, ch] = u1 + u2 + 0.5
        oi_ref[0, ch] = u3 - u1 + u2


@jax.jit
def kernel(Inp_AD_C_real, Inp_AD_C_imag):
    B, C, H, W = Inp_AD_C_real.shape
    G = _GROUPS
    chans = C // G

    img_spec = pl.BlockSpec((1, chans, H, W), lambda b, g: (b, g, 0, 0))
    row_spec = pl.BlockSpec((1, 1, W), lambda b, g: (2 * b + g, 0, 0))
    mat_spec = pl.BlockSpec((_N, _N), lambda b, g: (0, 0))

    energy = pl.pallas_call(
        _energy_kernel,
        grid=(B, G),
        in_specs=[img_spec, img_spec],
        out_specs=row_spec,
        out_shape=jax.ShapeDtypeStruct((_ROWS, 1, W), jnp.float32),
    )(Inp_AD_C_real, Inp_AD_C_imag)

    mask = _sc_mask(energy)

    out_r, out_i = pl.pallas_call(
        _transform_kernel,
        grid=(B, G),
        in_specs=[img_spec, img_spec, row_spec, mat_spec, mat_spec,
                  mat_spec, mat_spec],
        out_specs=[img_spec, img_spec],
        out_shape=[
            jax.ShapeDtypeStruct((B, C, H, W), jnp.float32),
            jax.ShapeDtypeStruct((B, C, H, W), jnp.float32),
        ],
    )(Inp_AD_C_real, Inp_AD_C_imag, mask,
      jnp.asarray(_COS, jnp.bfloat16), jnp.asarray(_SIN, jnp.bfloat16),
      jnp.asarray(_CPS, jnp.bfloat16), jnp.asarray(_CMS, jnp.bfloat16))

    return jax.lax.complex(out_r, out_i).astype(jnp.complex64)
